# baseline (speedup 1.0000x reference)
import functools

import jax
import jax.numpy as jnp
from jax.experimental import pallas as pl
from jax.experimental.pallas import tpu as pltpu

B = 2048
INPUT_DIM = 1024
N_CATEGORY = 1024
DIM_LATENT = 64
N_QUERY = 16
N_HIDDEN = 1024

BLK = 128
NB = B // BLK


_SQRT_HALF = 0.7071067811865476


def _gelu(v):
    return 0.5 * v * (1.0 + jax.lax.erf(v * _SQRT_HALF))


def _bf(v):
    return v.astype(jnp.bfloat16)


def _body(x_ref, etf_ref, etb_ref, eb_ref,
          w1_ref, b1_ref, w2_ref, b2_ref, w3_ref, b3_ref,
          w4_ref, b4_ref, w5_ref, b5_ref, w6_ref, b6_ref,
          xp_ref, zd_ref, s1_ref, s2_ref,
          w1s, w2s, w3s, w4s, w5s, w6s):
    f32 = jnp.float32
    i = pl.program_id(0)

    @pl.when(i == 0)
    def _cast_weights():
        w1s[...] = _bf(w1_ref[...])
        w2s[...] = _bf(w2_ref[...])
        w3s[...] = _bf(w3_ref[...])
        w4s[...] = _bf(w4_ref[...])
        w5s[...] = _bf(w5_ref[...])
        w6s[...] = _bf(w6_ref[...])

    xb = x_ref[...]
    h = _gelu(jnp.dot(_bf(xb), w1s[...], preferred_element_type=f32)
              + b1_ref[...])
    h = _gelu(jnp.dot(_bf(h), w2s[...], preferred_element_type=f32)
              + b2_ref[...])
    ze = jnp.dot(_bf(h), w3s[...], preferred_element_type=f32) + b3_ref[...]

    etf = etf_ref[...]
    enorm = jnp.sum(etf * etf, axis=0, keepdims=True)
    etb = etb_ref[...]
    eb = eb_ref[...]
    iota = jax.lax.broadcasted_iota(jnp.int32, (BLK, N_CATEGORY), 1)
    iota_kc = jax.lax.broadcasted_iota(jnp.int32, (BLK, 256), 1)

    KC = 256
    idx_list = []
    zq_list = []
    s2_part = jnp.zeros((), f32)
    for q in range(N_QUERY):
        ze_q = ze[:, q * DIM_LATENT:(q + 1) * DIM_LATENT]
        znq = jnp.sum(ze_q * ze_q, axis=1, keepdims=True)
        zeb = _bf(ze_q)
        best = None
        bidx = None
        for c0 in range(0, N_CATEGORY, KC):
            mat2 = jnp.dot(zeb, etb[:, c0:c0 + KC],
                           preferred_element_type=f32)
            d2c = znq + enorm[:, c0:c0 + KC] + mat2
            mnc = jnp.min(d2c, axis=1, keepdims=True)
            idxc = jnp.min(
                jnp.where(d2c == mnc, iota_kc, KC),
                axis=1, keepdims=True) + c0
            if best is None:
                best, bidx = mnc, idxc
            else:
                upd = mnc < best
                best = jnp.where(upd, mnc, best)
                bidx = jnp.where(upd, idxc, bidx)
        idx_list.append(bidx)
        ohm = iota == bidx
        zq_q = jnp.dot(ohm.astype(jnp.bfloat16), eb,
                       preferred_element_type=f32)
        zq_list.append(zq_q)
        diff = ze_q - zq_q
        s2_part = s2_part + jnp.sum(diff * diff)

    idxmat = jnp.concatenate(idx_list, axis=1)
    iota3 = jax.lax.broadcasted_iota(
        jnp.int32, (BLK, N_QUERY, N_CATEGORY), 2)
    zd_ref[...] = (iota3 == idxmat[:, :, None]).astype(jnp.int32)
    zqf = jnp.concatenate(zq_list, axis=1)

    h = _gelu(jnp.dot(_bf(zqf), w4s[...], preferred_element_type=f32)
              + b4_ref[...])
    h = _gelu(jnp.dot(_bf(h), w5s[...], preferred_element_type=f32)
              + b5_ref[...])
    xp = jax.nn.sigmoid(
        jnp.dot(_bf(h), w6s[...], preferred_element_type=f32)
        + b6_ref[...])
    xp_ref[...] = xp

    dx = xb - xp
    s1_ref[...] = jnp.sum(dx * dx).reshape(1, 1, 1)
    s2_ref[...] = s2_part.reshape(1, 1, 1)


@functools.partial(jax.jit, static_argnames=("interpret",))
def _run(x, embed_pool, W1, b1, W2, b2, W3, b3, W4, b4, W5, b5, W6, b6,
         interpret=False):
    f32 = jnp.float32
    etf = embed_pool.T
    etb = _bf(etf) * jnp.bfloat16(-2.0)
    eb = _bf(embed_pool)
    ws = [W1, W2, W3, W4, W5, W6]
    bs = [b.reshape(1, -1) for b in (b1, b2, b3, b4, b5, b6)]

    row_spec = lambda cols: pl.BlockSpec((BLK, cols), lambda i: (i, 0))
    full = lambda a: pl.BlockSpec(a.shape, lambda i: (0, 0))

    in_specs = [row_spec(INPUT_DIM), full(etf), full(etb), full(eb)]
    ops = []
    for w, bias in zip(ws, bs):
        in_specs += [full(w), full(bias)]
        ops += [w, bias]

    out_shape = [
        jax.ShapeDtypeStruct((B, INPUT_DIM), f32),
        jax.ShapeDtypeStruct((B, N_QUERY, N_CATEGORY), jnp.int32),
        jax.ShapeDtypeStruct((NB, 1, 1), f32),
        jax.ShapeDtypeStruct((NB, 1, 1), f32),
    ]
    out_specs = [
        row_spec(INPUT_DIM),
        pl.BlockSpec((BLK, N_QUERY, N_CATEGORY), lambda i: (i, 0, 0)),
        pl.BlockSpec((1, 1, 1), lambda i: (i, 0, 0)),
        pl.BlockSpec((1, 1, 1), lambda i: (i, 0, 0)),
    ]

    xp, zd, s1, s2 = pl.pallas_call(
        _body,
        grid=(NB,),
        in_specs=in_specs,
        out_specs=out_specs,
        out_shape=out_shape,
        compiler_params=pltpu.CompilerParams(
            dimension_semantics=("arbitrary",)),
        scratch_shapes=[pltpu.VMEM((N_HIDDEN, N_HIDDEN), jnp.bfloat16)
                        for _ in range(6)],
        interpret=interpret,
    )(x, etf, etb, eb, *ops)

    denom = f32(B * INPUT_DIM)
    loss = (jnp.sum(s1) / denom + 1.25 * (jnp.sum(s2) / denom)) / f32(B)
    return xp, zd, loss


def kernel(x, embed_pool, W1, b1, W2, b2, W3, b3, W4, b4, W5, b5, W6, b6):
    return _run(x, embed_pool, W1, b1, W2, b2, W3, b3, W4, b4,
                W5, b5, W6, b6)


# speedup vs baseline: 1.0229x; 1.0229x over previous
import functools

import jax
import jax.numpy as jnp
from jax.experimental import pallas as pl
from jax.experimental.pallas import tpu as pltpu

B = 2048
INPUT_DIM = 1024
N_CATEGORY = 1024
DIM_LATENT = 64
N_QUERY = 16
N_HIDDEN = 1024

BLK = 128
NB = B // BLK


_SQRT_HALF = 0.7071067811865476


def _gelu(v):
    return 0.5 * v * (1.0 + jax.lax.erf(v * _SQRT_HALF))


def _bf(v):
    return v.astype(jnp.bfloat16)


def _body(x_ref, etf_ref, etb_ref, eb_ref,
          w1_ref, b1_ref, w2_ref, b2_ref, w3_ref, b3_ref,
          w4_ref, b4_ref, w5_ref, b5_ref, w6_ref, b6_ref,
          xp_ref, zd_ref, loss_ref,
          s1_s, s2_s, w1s, w2s, w3s, w4s, w5s, w6s):
    f32 = jnp.float32
    i = pl.program_id(0)

    @pl.when(i == 0)
    def _cast_weights():
        s1_s[0, 0] = 0.0
        s2_s[0, 0] = 0.0
        w1s[...] = _bf(w1_ref[...])
        w2s[...] = _bf(w2_ref[...])
        w3s[...] = _bf(w3_ref[...])
        w4s[...] = _bf(w4_ref[...])
        w5s[...] = _bf(w5_ref[...])
        w6s[...] = _bf(w6_ref[...])

    xb = x_ref[...]
    h = _gelu(jnp.dot(_bf(xb), w1s[...], preferred_element_type=f32)
              + b1_ref[...])
    h = _gelu(jnp.dot(_bf(h), w2s[...], preferred_element_type=f32)
              + b2_ref[...])
    ze = jnp.dot(_bf(h), w3s[...], preferred_element_type=f32) + b3_ref[...]

    etf = etf_ref[...]
    enorm = jnp.sum(etf * etf, axis=0, keepdims=True)
    etb = etb_ref[...]
    eb = eb_ref[...]
    iota = jax.lax.broadcasted_iota(jnp.int32, (BLK, N_CATEGORY), 1)
    iota_kc = jax.lax.broadcasted_iota(jnp.int32, (BLK, 256), 1)

    KC = 256
    idx_list = []
    zq_list = []
    s2_part = jnp.zeros((), f32)
    for q in range(N_QUERY):
        ze_q = ze[:, q * DIM_LATENT:(q + 1) * DIM_LATENT]
        znq = jnp.sum(ze_q * ze_q, axis=1, keepdims=True)
        zeb = _bf(ze_q)
        best = None
        bidx = None
        for c0 in range(0, N_CATEGORY, KC):
            mat2 = jnp.dot(zeb, etb[:, c0:c0 + KC],
                           preferred_element_type=f32)
            d2c = znq + enorm[:, c0:c0 + KC] + mat2
            mnc = jnp.min(d2c, axis=1, keepdims=True)
            idxc = jnp.min(
                jnp.where(d2c == mnc, iota_kc, KC),
                axis=1, keepdims=True) + c0
            if best is None:
                best, bidx = mnc, idxc
            else:
                upd = mnc < best
                best = jnp.where(upd, mnc, best)
                bidx = jnp.where(upd, idxc, bidx)
        idx_list.append(bidx)
        ohm = iota == bidx
        zq_q = jnp.dot(ohm.astype(jnp.bfloat16), eb,
                       preferred_element_type=f32)
        zq_list.append(zq_q)
        diff = ze_q - zq_q
        s2_part = s2_part + jnp.sum(diff * diff)

    idxmat = jnp.concatenate(idx_list, axis=1)
    iota3 = jax.lax.broadcasted_iota(
        jnp.int32, (BLK, N_QUERY, N_CATEGORY), 2)
    zd_ref[...] = (iota3 == idxmat[:, :, None]).astype(jnp.int32)
    zqf = jnp.concatenate(zq_list, axis=1)

    h = _gelu(jnp.dot(_bf(zqf), w4s[...], preferred_element_type=f32)
              + b4_ref[...])
    h = _gelu(jnp.dot(_bf(h), w5s[...], preferred_element_type=f32)
              + b5_ref[...])
    xp = jax.nn.sigmoid(
        jnp.dot(_bf(h), w6s[...], preferred_element_type=f32)
        + b6_ref[...])
    xp_ref[...] = xp

    dx = xb - xp
    s1_s[0, 0] += jnp.sum(dx * dx)
    s2_s[0, 0] += s2_part

    @pl.when(i == NB - 1)
    def _loss():
        denom = f32(B * INPUT_DIM)
        loss = (s1_s[0, 0] / denom + 1.25 * (s2_s[0, 0] / denom)) / f32(B)
        loss_ref[...] = loss.reshape(1, 1)


@functools.partial(jax.jit, static_argnames=("interpret",))
def _run(x, embed_pool, W1, b1, W2, b2, W3, b3, W4, b4, W5, b5, W6, b6,
         interpret=False):
    f32 = jnp.float32
    etf = embed_pool.T
    etb = _bf(etf) * jnp.bfloat16(-2.0)
    eb = _bf(embed_pool)
    ws = [W1, W2, W3, W4, W5, W6]
    bs = [b.reshape(1, -1) for b in (b1, b2, b3, b4, b5, b6)]

    row_spec = lambda cols: pl.BlockSpec((BLK, cols), lambda i: (i, 0))
    full = lambda a: pl.BlockSpec(a.shape, lambda i: (0, 0))

    in_specs = [row_spec(INPUT_DIM), full(etf), full(etb), full(eb)]
    ops = []
    for w, bias in zip(ws, bs):
        in_specs += [full(w), full(bias)]
        ops += [w, bias]

    out_shape = [
        jax.ShapeDtypeStruct((B, INPUT_DIM), f32),
        jax.ShapeDtypeStruct((B, N_QUERY, N_CATEGORY), jnp.int32),
        jax.ShapeDtypeStruct((1, 1), f32),
    ]
    out_specs = [
        row_spec(INPUT_DIM),
        pl.BlockSpec((BLK, N_QUERY, N_CATEGORY), lambda i: (i, 0, 0)),
        pl.BlockSpec((1, 1), lambda i: (0, 0)),
    ]

    xp, zd, lossm = pl.pallas_call(
        _body,
        grid=(NB,),
        in_specs=in_specs,
        out_specs=out_specs,
        out_shape=out_shape,
        compiler_params=pltpu.CompilerParams(
            dimension_semantics=("arbitrary",)),
        scratch_shapes=[pltpu.SMEM((1, 1), jnp.float32)] * 2
        + [pltpu.VMEM((N_HIDDEN, N_HIDDEN), jnp.bfloat16)
           for _ in range(6)],
        interpret=interpret,
    )(x, etf, etb, eb, *ops)

    return xp, zd, lossm[0, 0]


def kernel(x, embed_pool, W1, b1, W2, b2, W3, b3, W4, b4, W5, b5, W6, b6):
    return _run(x, embed_pool, W1, b1, W2, b2, W3, b3, W4, b4,
                W5, b5, W6, b6)
